# baseline (speedup 1.0000x reference)
import jax
import jax.numpy as jnp
from jax import lax
from jax.experimental import pallas as pl
from jax.experimental.pallas import tpu as pltpu
from jax.experimental.pallas import tpu_sc as plsc

_SCALE_HALF = 32.0
_SIDE = 35
_V = _SIDE ** 3
_VPAD = 336 * 128
_QUARTER = _VPAD // 4
_ROW = 16384 * 3
_CHUNK = _ROW // 4
_NGROUP = _CHUNK // 48


def _sc_body(pred_hbm, gt_hbm, out_pred, out_gt,
             buf1, buf2, priv, minv, parts, slots, smin, sem1, sem2):
    c = lax.axis_index("c")
    s = lax.axis_index("s")
    base = s * _CHUNK

    @pl.when(c == 0)
    def _():
        pltpu.async_copy(pred_hbm.at[pl.ds(base, _CHUNK)], buf1, sem1)
        pltpu.async_copy(gt_hbm.at[pl.ds(base, _CHUNK)], buf2, sem2)

    @pl.when(c == 1)
    def _():
        pltpu.async_copy(gt_hbm.at[pl.ds(base, _CHUNK)], buf1, sem1)
        pltpu.async_copy(pred_hbm.at[pl.ds(base, _CHUNK)], buf2, sem2)

    zeros16 = jnp.zeros((16,), jnp.float32)

    def _zero(j, carry):
        priv[pl.ds(j * 16, 16)] = zeros16
        return carry

    lax.fori_loop(0, _VPAD // 16, _zero, 0, unroll=16)

    pltpu.make_async_copy(pred_hbm.at[pl.ds(base, _CHUNK)], buf1, sem1).wait()
    pltpu.make_async_copy(pred_hbm.at[pl.ds(base, _CHUNK)], buf2, sem2).wait()

    lanes = lax.iota(jnp.int32, 16)
    inf16 = jnp.full((16,), jnp.inf, jnp.float32)

    def _minbody(i, accs):
        a0, a1, a2 = accs
        o = i * 48
        a0 = jnp.minimum(a0, jnp.minimum(buf1[pl.ds(o, 16)],
                                         buf2[pl.ds(o, 16)]))
        a1 = jnp.minimum(a1, jnp.minimum(buf1[pl.ds(o + 16, 16)],
                                         buf2[pl.ds(o + 16, 16)]))
        a2 = jnp.minimum(a2, jnp.minimum(buf1[pl.ds(o + 32, 16)],
                                         buf2[pl.ds(o + 32, 16)]))
        return a0, a1, a2

    accs = lax.fori_loop(0, _NGROUP, _minbody, (inf16, inf16, inf16),
                         unroll=8)

    gdn = lax.GatherDimensionNumbers(offset_dims=(), collapsed_slice_dims=(0,),
                                     start_index_map=(0,))

    def _perm(v, idx):
        return lax.gather(v, idx.reshape(16, 1), gdn, slice_sizes=(1,),
                          mode=lax.GatherScatterMode.PROMISE_IN_BOUNDS)

    def _bmin(v):
        for k in (8, 4, 2, 1):
            v = jnp.minimum(v, _perm(v, jnp.bitwise_xor(lanes, k)))
        return v

    def _axis_min(a):
        v = inf16
        for r in range(3):
            msk = ((lanes + r) % 3) == a
            v = jnp.minimum(v, jnp.where(msk, accs[r], inf16))
        return _bmin(v)

    minv[pl.ds(0, 16)] = _axis_min(0)
    minv[pl.ds(16, 16)] = _axis_min(1)
    minv[pl.ds(32, 16)] = _axis_min(2)
    pltpu.sync_copy(minv, smin.at[pl.ds(s * 48, 48)])
    plsc.subcore_barrier()
    pltpu.sync_copy(smin, parts)

    def _pbody(r, accs):
        ax, ay, az = accs
        o = r * 48
        ax = jnp.minimum(ax, parts[pl.ds(o, 16)])
        ay = jnp.minimum(ay, parts[pl.ds(o + 16, 16)])
        az = jnp.minimum(az, parts[pl.ds(o + 32, 16)])
        return ax, ay, az

    gx, gy, gz = lax.fori_loop(0, 16, _pbody, (inf16, inf16, inf16))
    mnx = (gx * _SCALE_HALF).astype(jnp.int32) - 1
    mny = (gy * _SCALE_HALF).astype(jnp.int32) - 1
    mnz = (gz * _SCALE_HALF).astype(jnp.int32) - 1

    idx0 = lanes * 3
    one16 = jnp.full((16,), 1.0, jnp.float32)

    def _sbody(i, carry):
        o = i * 48 + idx0
        x = plsc.load_gather(buf1, [o]) * _SCALE_HALF
        y = plsc.load_gather(buf1, [o + 1]) * _SCALE_HALF
        z = plsc.load_gather(buf1, [o + 2]) * _SCALE_HALF
        ix = x.astype(jnp.int32)
        iy = y.astype(jnp.int32)
        iz = z.astype(jnp.int32)
        fx = x - ix.astype(jnp.float32)
        fy = y - iy.astype(jnp.float32)
        fz = z - iz.astype(jnp.float32)
        m = jnp.where((x + y + z) != 0.0, one16, zeros16)
        wx1 = fx * m
        wx0 = m - wx1
        wy0 = one16 - fy
        wz0 = one16 - fz
        w00 = wy0 * wz0
        w01 = wy0 * fz
        w10 = fy * wz0
        w11 = fy * fz
        flat = ((ix - mnx) * _SIDE + (iy - mny)) * _SIDE + (iz - mnz)
        wyz = (w00, w01, w10, w11)
        for dx in range(2):
            wx = wx1 if dx else wx0
            for dy in range(2):
                for dz in range(2):
                    off = dx * _SIDE * _SIDE + dy * _SIDE + dz
                    plsc.addupdate_scatter(priv, [flat + off],
                                           wx * wyz[dy * 2 + dz])
        return carry

    lax.fori_loop(0, _NGROUP, _sbody, 0, unroll=4)

    pltpu.sync_copy(priv, slots.at[pl.ds(s * _VPAD, _VPAD)])
    plsc.subcore_barrier()
    g = s // 4
    q = s % 4
    srcbase = g * 4 * _VPAD + q * _QUARTER
    for k in range(4):
        pltpu.sync_copy(slots.at[pl.ds(srcbase + k * _VPAD, _QUARTER)],
                        priv.at[pl.ds(k * _QUARTER, _QUARTER)])

    def _rbody(t, carry):
        o = t * 16
        v = ((priv[pl.ds(o, 16)] + priv[pl.ds(_QUARTER + o, 16)]) +
             (priv[pl.ds(2 * _QUARTER + o, 16)] +
              priv[pl.ds(3 * _QUARTER + o, 16)]))
        priv[pl.ds(o, 16)] = v
        return carry

    lax.fori_loop(0, _QUARTER // 16, _rbody, 0, unroll=16)

    @pl.when(c == 0)
    def _():
        pltpu.sync_copy(priv.at[pl.ds(0, _QUARTER)],
                        out_pred.at[g, pl.ds(q * _QUARTER, _QUARTER)])

    @pl.when(c == 1)
    def _():
        pltpu.sync_copy(priv.at[pl.ds(0, _QUARTER)],
                        out_gt.at[g, pl.ds(q * _QUARTER, _QUARTER)])


@jax.jit
def _gridding_sc(pred_flat, gt_flat):
    mesh = plsc.VectorSubcoreMesh(core_axis_name="c", subcore_axis_name="s")
    f = pl.kernel(
        _sc_body,
        out_type=(jax.ShapeDtypeStruct((4, _VPAD), jnp.float32),
                  jax.ShapeDtypeStruct((4, _VPAD), jnp.float32)),
        mesh=mesh,
        compiler_params=pltpu.CompilerParams(needs_layout_passes=False),
        scratch_types=[
            pltpu.VMEM((_CHUNK,), jnp.float32),
            pltpu.VMEM((_CHUNK,), jnp.float32),
            pltpu.VMEM((_VPAD,), jnp.float32),
            pltpu.VMEM((48,), jnp.float32),
            pltpu.VMEM((768,), jnp.float32),
            pltpu.VMEM_SHARED((16 * _VPAD,), jnp.float32),
            pltpu.VMEM_SHARED((768,), jnp.float32),
            pltpu.SemaphoreType.DMA,
            pltpu.SemaphoreType.DMA,
        ],
    )
    return f(pred_flat, gt_flat)


def kernel(pred_cloud, gt_cloud):
    pred = pred_cloud.reshape(-1)
    gt = gt_cloud.reshape(-1)
    out_pred, out_gt = _gridding_sc(pred, gt)
    return out_pred[:, :_V], out_gt[:, :_V]


# speedup vs baseline: 1.0066x; 1.0066x over previous
import jax
import jax.numpy as jnp
from jax import lax
from jax.experimental import pallas as pl
from jax.experimental.pallas import tpu as pltpu
from jax.experimental.pallas import tpu_sc as plsc

_SCALE_HALF = 32.0
_SIDE = 35
_V = _SIDE ** 3
_VPAD = 336 * 128
_QUARTER = _VPAD // 4
_ROW = 16384 * 3
_CHUNK = _ROW // 4
_NGROUP = _CHUNK // 48


def _sc_body(pred_hbm, gt_hbm, out_pred, out_gt,
             buf1, buf2, priv, minv, parts, slots, smin, sem1, sem2):
    c = lax.axis_index("c")
    s = lax.axis_index("s")
    base = s * _CHUNK

    @pl.when(c == 0)
    def _():
        pltpu.async_copy(pred_hbm.at[pl.ds(base, _CHUNK)], buf1, sem1)
        pltpu.async_copy(gt_hbm.at[pl.ds(base, _CHUNK)], buf2, sem2)

    @pl.when(c == 1)
    def _():
        pltpu.async_copy(gt_hbm.at[pl.ds(base, _CHUNK)], buf1, sem1)
        pltpu.async_copy(pred_hbm.at[pl.ds(base, _CHUNK)], buf2, sem2)

    zeros16 = jnp.zeros((16,), jnp.float32)

    def _zero(j, carry):
        priv[pl.ds(j * 16, 16)] = zeros16
        return carry

    lax.fori_loop(0, _VPAD // 16, _zero, 0, unroll=16)

    lanes = lax.iota(jnp.int32, 16)
    inf16 = jnp.full((16,), jnp.inf, jnp.float32)

    def _minbody_for(buf):
        def _minbody(i, accs):
            a0, a1, a2 = accs
            o = i * 48
            a0 = jnp.minimum(a0, buf[pl.ds(o, 16)])
            a1 = jnp.minimum(a1, buf[pl.ds(o + 16, 16)])
            a2 = jnp.minimum(a2, buf[pl.ds(o + 32, 16)])
            return a0, a1, a2
        return _minbody

    pltpu.make_async_copy(pred_hbm.at[pl.ds(base, _CHUNK)], buf1, sem1).wait()
    accs = lax.fori_loop(0, _NGROUP, _minbody_for(buf1),
                         (inf16, inf16, inf16), unroll=8)
    pltpu.make_async_copy(pred_hbm.at[pl.ds(base, _CHUNK)], buf2, sem2).wait()
    accs = lax.fori_loop(0, _NGROUP, _minbody_for(buf2), accs, unroll=8)

    gdn = lax.GatherDimensionNumbers(offset_dims=(), collapsed_slice_dims=(0,),
                                     start_index_map=(0,))

    def _perm(v, idx):
        return lax.gather(v, idx.reshape(16, 1), gdn, slice_sizes=(1,),
                          mode=lax.GatherScatterMode.PROMISE_IN_BOUNDS)

    def _bmin(v):
        for k in (8, 4, 2, 1):
            v = jnp.minimum(v, _perm(v, jnp.bitwise_xor(lanes, k)))
        return v

    def _axis_min(a):
        v = inf16
        for r in range(3):
            msk = ((lanes + r) % 3) == a
            v = jnp.minimum(v, jnp.where(msk, accs[r], inf16))
        return _bmin(v)

    minv[pl.ds(0, 16)] = _axis_min(0)
    minv[pl.ds(16, 16)] = _axis_min(1)
    minv[pl.ds(32, 16)] = _axis_min(2)
    pltpu.sync_copy(minv, smin.at[pl.ds(s * 48, 48)])
    plsc.subcore_barrier()
    pltpu.sync_copy(smin, parts)

    def _pbody(r, accs):
        ax, ay, az = accs
        o = r * 48
        ax = jnp.minimum(ax, parts[pl.ds(o, 16)])
        ay = jnp.minimum(ay, parts[pl.ds(o + 16, 16)])
        az = jnp.minimum(az, parts[pl.ds(o + 32, 16)])
        return ax, ay, az

    gx, gy, gz = lax.fori_loop(0, 16, _pbody, (inf16, inf16, inf16))
    mnx = (gx * _SCALE_HALF).astype(jnp.int32) - 1
    mny = (gy * _SCALE_HALF).astype(jnp.int32) - 1
    mnz = (gz * _SCALE_HALF).astype(jnp.int32) - 1

    idx0 = lanes * 3
    one16 = jnp.full((16,), 1.0, jnp.float32)

    def _sbody(i, carry):
        o = i * 48 + idx0
        x = plsc.load_gather(buf1, [o]) * _SCALE_HALF
        y = plsc.load_gather(buf1, [o + 1]) * _SCALE_HALF
        z = plsc.load_gather(buf1, [o + 2]) * _SCALE_HALF
        ix = x.astype(jnp.int32)
        iy = y.astype(jnp.int32)
        iz = z.astype(jnp.int32)
        fx = x - ix.astype(jnp.float32)
        fy = y - iy.astype(jnp.float32)
        fz = z - iz.astype(jnp.float32)
        m = jnp.where((x + y + z) != 0.0, one16, zeros16)
        wx1 = fx * m
        wx0 = m - wx1
        wy0 = one16 - fy
        wz0 = one16 - fz
        w00 = wy0 * wz0
        w01 = wy0 * fz
        w10 = fy * wz0
        w11 = fy * fz
        flat = ((ix - mnx) * _SIDE + (iy - mny)) * _SIDE + (iz - mnz)
        wyz = (w00, w01, w10, w11)
        for dx in range(2):
            wx = wx1 if dx else wx0
            for dy in range(2):
                for dz in range(2):
                    off = dx * _SIDE * _SIDE + dy * _SIDE + dz
                    plsc.addupdate_scatter(priv, [flat + off],
                                           wx * wyz[dy * 2 + dz])
        return carry

    lax.fori_loop(0, _NGROUP, _sbody, 0, unroll=4)

    g = s // 4
    q = s % 4
    for j in range(1, 4):
        k = (q + j) % 4
        pltpu.sync_copy(priv.at[pl.ds(k * _QUARTER, _QUARTER)],
                        slots.at[pl.ds((s * 4 + k) * _QUARTER, _QUARTER)])
    plsc.subcore_barrier()
    for j in range(1, 4):
        peer = g * 4 + (q + j) % 4
        dst = ((q + j) % 4) * _QUARTER
        pltpu.sync_copy(
            slots.at[pl.ds((peer * 4 + q) * _QUARTER, _QUARTER)],
            priv.at[pl.ds(dst, _QUARTER)])

    def _rbody(t, carry):
        o = t * 16
        v = ((priv[pl.ds(o, 16)] + priv[pl.ds(_QUARTER + o, 16)]) +
             (priv[pl.ds(2 * _QUARTER + o, 16)] +
              priv[pl.ds(3 * _QUARTER + o, 16)]))
        priv[pl.ds(o, 16)] = v
        return carry

    lax.fori_loop(0, _QUARTER // 16, _rbody, 0, unroll=16)

    @pl.when(c == 0)
    def _():
        pltpu.sync_copy(priv.at[pl.ds(0, _QUARTER)],
                        out_pred.at[g, pl.ds(q * _QUARTER, _QUARTER)])

    @pl.when(c == 1)
    def _():
        pltpu.sync_copy(priv.at[pl.ds(0, _QUARTER)],
                        out_gt.at[g, pl.ds(q * _QUARTER, _QUARTER)])


@jax.jit
def _gridding_sc(pred_flat, gt_flat):
    mesh = plsc.VectorSubcoreMesh(core_axis_name="c", subcore_axis_name="s")
    f = pl.kernel(
        _sc_body,
        out_type=(jax.ShapeDtypeStruct((4, _VPAD), jnp.float32),
                  jax.ShapeDtypeStruct((4, _VPAD), jnp.float32)),
        mesh=mesh,
        compiler_params=pltpu.CompilerParams(needs_layout_passes=False),
        scratch_types=[
            pltpu.VMEM((_CHUNK,), jnp.float32),
            pltpu.VMEM((_CHUNK,), jnp.float32),
            pltpu.VMEM((_VPAD,), jnp.float32),
            pltpu.VMEM((48,), jnp.float32),
            pltpu.VMEM((768,), jnp.float32),
            pltpu.VMEM_SHARED((16 * _VPAD,), jnp.float32),
            pltpu.VMEM_SHARED((768,), jnp.float32),
            pltpu.SemaphoreType.DMA,
            pltpu.SemaphoreType.DMA,
        ],
    )
    return f(pred_flat, gt_flat)


def kernel(pred_cloud, gt_cloud):
    pred = pred_cloud.reshape(-1)
    gt = gt_cloud.reshape(-1)
    out_pred, out_gt = _gridding_sc(pred, gt)
    return out_pred[:, :_V], out_gt[:, :_V]
